# Initial kernel scaffold; baseline (speedup 1.0000x reference)
import jax
import jax.numpy as jnp
from jax.experimental import pallas as pl


def kernel(points_coor, points_fea, W):
    raise NotImplementedError("write your pallas kernel here")


# speedup vs baseline: 1.6850x; 1.6850x over previous
import functools

import jax
import jax.numpy as jnp
from jax.experimental import pallas as pl
from jax.experimental.pallas import tpu as pltpu

_B, _N, _C = 4, 16384, 64
_S, _RADIUS, _G = 512, 0.2, 32
_R, _L = 128, 128


def _fps_body(coor_ref, out_ref, dist_ref):
    x = coor_ref[:, 0, :, :]
    y = coor_ref[:, 1, :, :]
    z = coor_ref[:, 2, :, :]
    dist_ref[...] = jnp.full((_B, _R, _L), 1e10, dtype=jnp.float32)

    iota_r = jax.lax.broadcasted_iota(jnp.int32, (_B, _R, _L), 1)
    iota_l = jax.lax.broadcasted_iota(jnp.int32, (_B, _R, _L), 2)
    gidx = iota_r * _L + iota_l
    out_lane = jax.lax.broadcasted_iota(jnp.int32, (_B, 3, _S), 2)

    def body(i, farthest):
        sel = gidx == farthest
        cx = jnp.sum(jnp.where(sel, x, 0.0), axis=(1, 2), keepdims=True)
        cy = jnp.sum(jnp.where(sel, y, 0.0), axis=(1, 2), keepdims=True)
        cz = jnp.sum(jnp.where(sel, z, 0.0), axis=(1, 2), keepdims=True)
        cvec = jnp.concatenate([cx, cy, cz], axis=1)
        out_ref[...] = jnp.where(out_lane == i,
                                 jnp.broadcast_to(cvec, (_B, 3, _S)),
                                 out_ref[...])
        dx = x - cx
        dy = y - cy
        dz = z - cz
        d = jnp.sqrt((dx * dx + dy * dy) + dz * dz + 1e-12)
        dist = jnp.minimum(dist_ref[...], d)
        dist_ref[...] = dist
        m = jnp.max(dist, axis=(1, 2), keepdims=True)
        cand = jnp.where(dist == m, gidx, jnp.int32(2**31 - 1))
        return jnp.min(cand, axis=(1, 2), keepdims=True)

    jax.lax.fori_loop(0, _S, body, jnp.zeros((_B, 1, 1), jnp.int32))


@jax.jit
def _fps_new_coor(points_coor):
    coor4 = points_coor.reshape(_B, 3, _R, _L)
    return pl.pallas_call(
        _fps_body,
        out_shape=jax.ShapeDtypeStruct((_B, 3, _S), jnp.float32),
        scratch_shapes=[pltpu.VMEM((_B, _R, _L), jnp.float32)],
    )(coor4)


def _index_points(points, idx):
    b = points.shape[0]
    if idx.ndim == 2:
        return points[jnp.arange(b)[:, None], idx]
    return points[jnp.arange(b)[:, None, None], idx]


def kernel(points_coor, points_fea, W):
    new_coor_out = _fps_new_coor(points_coor)

    pc = jnp.transpose(points_coor, (0, 2, 1))
    pf = jnp.transpose(points_fea, (0, 2, 1))
    new_coor = jnp.transpose(new_coor_out, (0, 2, 1))

    d = -2.0 * jnp.matmul(new_coor, jnp.swapaxes(pc, 1, 2))
    d = d + jnp.sum(new_coor ** 2, axis=-1)[:, :, None]
    d = d + jnp.sum(pc ** 2, axis=-1)[:, None, :]
    neg_d, group_idx = jax.lax.top_k(-d, _G)
    dist_k = -neg_d
    mask = dist_k > _RADIUS ** 2
    group_first = jnp.broadcast_to(group_idx[:, :, :1], group_idx.shape)
    group_idx = jnp.where(mask, group_first, group_idx)

    gpc = _index_points(pc, group_idx)
    gpc = (gpc - new_coor[:, :, None, :]) / _RADIUS
    gpf = _index_points(pf, group_idx)
    gpf = jnp.concatenate([gpf, gpc], axis=-1)
    xall = jnp.transpose(gpf, (0, 3, 2, 1))
    xall = jnp.einsum('oc,bcgs->bogs', W, xall)
    xall = jax.nn.relu(xall)
    new_fea = jnp.max(xall, axis=2)
    return (new_coor_out, new_fea)
